# baseline (speedup 1.0000x reference)
import functools

import jax
import jax.numpy as jnp
from jax import lax
from jax.experimental import pallas as pl
from jax.experimental.pallas import tpu as pltpu
from jax.experimental.pallas import tpu_sc as plsc

_NC, _NS = 2, 16
_NW = _NC * _NS
_L16 = 16


@functools.cache
def _build_compact(v, d):
    blk = 1000
    nblk = v // 2 // blk

    def body(lo_ref, hi_ref, out_ref):
        out_ref[:, :d] = lo_ref[...]
        out_ref[:, d:] = hi_ref[...]

    return pl.pallas_call(
        body,
        grid=(nblk,),
        in_specs=[
            pl.BlockSpec((blk, d), lambda i: (i, 0)),
            pl.BlockSpec((blk, d), lambda i: (i + nblk, 0)),
        ],
        out_specs=pl.BlockSpec((blk, 2 * d), lambda i: (i, 0)),
        out_shape=jax.ShapeDtypeStruct((v // 2, 2 * d), jnp.float32),
    )


@functools.cache
def _build_gather(b, l, d, half):
    n = b * l
    seq_per_w = b // _NW
    cpad = l + (-l % _L16)
    n_groups = d // _L16
    mesh = plsc.VectorSubcoreMesh(core_axis_name="c", subcore_axis_name="s")

    @functools.partial(
        pl.kernel,
        out_type=jax.ShapeDtypeStruct((b, l, d), jnp.float32),
        mesh=mesh,
        scratch_types=[
            pltpu.VMEM((cpad,), jnp.int32),
            pltpu.VMEM((cpad,), jnp.int32),
            pltpu.VMEM((cpad,), jnp.int32),
            pltpu.VMEM((cpad,), jnp.int32),
            pltpu.VMEM((cpad, 2 * d), jnp.float32),
            pltpu.VMEM((cpad, 2 * d), jnp.float32),
            pltpu.VMEM((l, d), jnp.float32),
            pltpu.VMEM((l, d), jnp.float32),
            pltpu.SemaphoreType.DMA,
            pltpu.SemaphoreType.DMA,
            pltpu.SemaphoreType.DMA,
            pltpu.SemaphoreType.DMA,
            pltpu.SemaphoreType.DMA,
            pltpu.SemaphoreType.DMA,
        ],
        compiler_params=pltpu.CompilerParams(needs_layout_passes=False),
    )
    def gather(idx_hbm, table2_hbm, out_hbm,
               idx0, idx1, pidx0, pidx1, rows0, rows1, sel0, sel1,
               isem0, isem1, gsem0, gsem1, wsem0, wsem1):
        wid = lax.axis_index("s") * _NC + lax.axis_index("c")
        seq0 = wid * seq_per_w
        idx_v = (idx0, idx1)
        pidx_v = (pidx0, pidx1)
        rows_v = (rows0, rows1)
        sel_v = (sel0, sel1)
        isem = (isem0, isem1)
        gsem = (gsem0, gsem1)
        wsem = (wsem0, wsem1)
        lanes = lax.iota(jnp.int32, _L16)

        def idx_start(i, u):
            pltpu.async_copy(
                idx_hbm.at[pl.ds((seq0 + i) * l, l)],
                idx_v[u].at[pl.ds(0, l)], isem[u])

        def idx_wait(u):
            pltpu.make_async_copy(
                idx_hbm.at[pl.ds(0, l)], idx_v[u].at[pl.ds(0, l)],
                isem[u]).wait()

        def pidx_compute(u):
            @pl.loop(0, cpad // _L16)
            def _(j):
                tok = idx_v[u][pl.ds(j * _L16, _L16)]
                vals = jnp.where(tok >= half, tok - half, tok)
                valid = (j * _L16 + lanes) < l
                pidx_v[u][pl.ds(j * _L16, _L16)] = jnp.where(valid, vals, 0)

        def gather_start(u):
            pltpu.async_copy(table2_hbm.at[pidx_v[u]], rows_v[u], gsem[u])

        def gather_wait(u):
            pltpu.make_async_copy(
                table2_hbm.at[pidx_v[u]], rows_v[u], gsem[u]).wait()

        def select(u):
            @pl.loop(0, l)
            def _(i):
                row = jnp.full((_L16,), i, dtype=jnp.int32)
                tok = plsc.load_gather(idx_v[u], [row])
                off = jnp.where(tok >= half, d, 0)
                for g in range(n_groups):
                    col = off + g * _L16 + lanes
                    vals = plsc.load_gather(rows_v[u], [row, col])
                    plsc.store_scatter(sel_v[u], [row, g * _L16 + lanes], vals)

        def write_start(i, u):
            pltpu.async_copy(sel_v[u], out_hbm.at[seq0 + i], wsem[u])

        def write_wait(u):
            pltpu.make_async_copy(
                sel_v[u], out_hbm.at[0], wsem[u]).wait()

        idx_start(0, 0)
        idx_start(1, 1)
        idx_wait(0)
        pidx_compute(0)
        gather_start(0)

        @pl.loop(0, seq_per_w // 2)
        def _outer(j):
            for u in (0, 1):
                i = j * 2 + u
                nu = 1 - u
                gather_wait(u)

                @pl.when(i + 1 < seq_per_w)
                def _():
                    idx_wait(nu)
                    pidx_compute(nu)
                    gather_start(nu)

                @pl.when(i >= 2)
                def _():
                    write_wait(u)
                select(u)

                @pl.when(i + 2 < seq_per_w)
                def _():
                    idx_start(i + 2, u)

                write_start(i, u)

        write_wait(0)
        write_wait(1)

    return gather


def kernel(sequences_tokens, embedding_table):
    b, l = sequences_tokens.shape
    v, d = embedding_table.shape
    idx = sequences_tokens.reshape(b * l)
    table2 = _build_compact(v, d)(embedding_table, embedding_table)
    return _build_gather(b, l, d, v // 2)(idx, table2)
